# baseline (speedup 1.0000x reference)
import functools

import jax
import jax.numpy as jnp
from jax import lax
from jax.experimental import pallas as pl
from jax.experimental.pallas import tpu as pltpu
from jax.experimental.pallas import tpu_sc as plsc


def _build(B, L, V, D, NC, NS):
    NW = NC * NS
    ROWS = B * L
    rows_per_w = ROWS // NW
    C = 2 * L
    n_chunks = rows_per_w // C
    SUB = 80
    n_sub = C // SUB

    mesh = plsc.VectorSubcoreMesh(core_axis_name="c", subcore_axis_name="s")

    @functools.partial(
        pl.kernel,
        out_type=jax.ShapeDtypeStruct((ROWS, D), jnp.float32),
        mesh=mesh,
        compiler_params=pltpu.CompilerParams(use_tc_tiling_on_sc=False),
        scratch_types=[
            pltpu.VMEM((C,), jnp.int32),
            pltpu.VMEM((C, D), jnp.float32),
            pltpu.VMEM_SHARED((C, D), jnp.float32),
            pltpu.SemaphoreType.DMA,
        ],
    )
    def embed(idx_hbm, tok_hbm, pos_hbm, out_hbm, idx_v, buf_v, pat_sh, sem):
        wid = lax.axis_index("s") * NC + lax.axis_index("c")
        wbase = wid * rows_per_w

        @pl.when(lax.axis_index("s") == 0)
        def _stage():
            pltpu.sync_copy(pos_hbm, buf_v.at[pl.ds(0, L)])
            pltpu.sync_copy(buf_v.at[pl.ds(0, L)], pat_sh.at[pl.ds(0, L)])
            pltpu.sync_copy(buf_v.at[pl.ds(0, L)], pat_sh.at[pl.ds(L, L)])

        plsc.subcore_barrier()

        @pl.loop(0, n_chunks)
        def _chunk(i):
            base = wbase + i * C
            pltpu.sync_copy(idx_hbm.at[pl.ds(base, C)], idx_v)
            pltpu.sync_copy(pat_sh, buf_v)
            descs = []
            for j in range(n_sub):
                descs.append(pltpu.async_copy(
                    tok_hbm.at[idx_v.at[pl.ds(j * SUB, SUB)]],
                    buf_v.at[pl.ds(j * SUB, SUB)],
                    sem, add=True))
            for d in descs:
                d.wait()
            pltpu.sync_copy(buf_v, out_hbm.at[pl.ds(base, C)])

    return embed


def kernel(x, token_table, pos_table):
    B, L = x.shape
    V, D = token_table.shape
    try:
        info = plsc.get_sparse_core_info()
        NC, NS = info.num_cores, info.num_subcores
    except Exception:
        NC, NS = 2, 16
    xf = x.reshape(-1).astype(jnp.int32)
    out = _build(B, L, V, D, NC, NS)(xf, token_table, pos_table)
    return out.reshape(B, L, D)


# speedup vs baseline: 1.0536x; 1.0536x over previous
import functools

import jax
import jax.numpy as jnp
from jax import lax
from jax.experimental import pallas as pl
from jax.experimental.pallas import tpu as pltpu
from jax.experimental.pallas import tpu_sc as plsc


def _build(B, L, V, D, NC, NS):
    NW = NC * NS
    ROWS = B * L
    rows_per_w = ROWS // NW
    C = 4 * L
    n_chunks = rows_per_w // C
    SUB = 80
    n_sub = C // SUB

    mesh = plsc.VectorSubcoreMesh(core_axis_name="c", subcore_axis_name="s")

    @functools.partial(
        pl.kernel,
        out_type=jax.ShapeDtypeStruct((ROWS, D), jnp.float32),
        mesh=mesh,
        compiler_params=pltpu.CompilerParams(use_tc_tiling_on_sc=False),
        scratch_types=[
            pltpu.VMEM((2, C), jnp.int32),
            pltpu.VMEM((2, C, D), jnp.float32),
            pltpu.VMEM_SHARED((C, D), jnp.float32),
            pltpu.SemaphoreType.DMA,
            pltpu.SemaphoreType.DMA,
            pltpu.SemaphoreType.DMA,
            pltpu.SemaphoreType.DMA,
        ],
    )
    def embed(idx_hbm, tok_hbm, pos_hbm, out_hbm,
              idx_v, buf_v, pat_sh, sem_g0, sem_g1, sem_w0, sem_w1):
        sem_g = (sem_g0, sem_g1)
        sem_w = (sem_w0, sem_w1)
        wid = lax.axis_index("s") * NC + lax.axis_index("c")
        wbase = wid * rows_per_w

        @pl.when(lax.axis_index("s") == 0)
        def _stage():
            pltpu.sync_copy(pos_hbm, buf_v.at[0, pl.ds(0, L)])
            for r in range(C // L):
                pltpu.sync_copy(buf_v.at[0, pl.ds(0, L)],
                                pat_sh.at[pl.ds(r * L, L)])

        plsc.subcore_barrier()

        def fire(i, slot):
            base = wbase + i * C
            pltpu.sync_copy(idx_hbm.at[pl.ds(base, C)], idx_v.at[slot])
            pltpu.sync_copy(pat_sh, buf_v.at[slot])
            for j in range(n_sub):
                pltpu.async_copy(
                    tok_hbm.at[idx_v.at[slot, pl.ds(j * SUB, SUB)]],
                    buf_v.at[slot, pl.ds(j * SUB, SUB)],
                    sem_g[slot], add=True)

        def wait_gathers(slot):
            for j in range(n_sub):
                pltpu.make_async_copy(
                    tok_hbm.at[idx_v.at[slot, pl.ds(j * SUB, SUB)]],
                    buf_v.at[slot, pl.ds(j * SUB, SUB)],
                    sem_g[slot]).wait()

        def wb_wait(i, slot):
            pltpu.make_async_copy(
                buf_v.at[slot],
                out_hbm.at[pl.ds(wbase + i * C, C)],
                sem_w[slot]).wait()

        fire(0, 0)

        @pl.loop(0, n_chunks // 2)
        def _groups(g):
            for b in (0, 1):
                i = 2 * g + b
                nb = (b + 1) % 2

                @pl.when(i + 1 < n_chunks)
                def _next():
                    @pl.when(i >= 1)
                    def _drain():
                        wb_wait(i - 1, nb)
                    fire(i + 1, nb)

                wait_gathers(b)
                pltpu.async_copy(
                    buf_v.at[b],
                    out_hbm.at[pl.ds(wbase + i * C, C)],
                    sem_w[b])

        for slot, i in ((0, n_chunks - 2), (1, n_chunks - 1)):
            wb_wait(i, slot)

    return embed


def kernel(x, token_table, pos_table):
    B, L = x.shape
    V, D = token_table.shape
    try:
        info = plsc.get_sparse_core_info()
        NC, NS = info.num_cores, info.num_subcores
    except Exception:
        NC, NS = 2, 16
    xf = x.reshape(-1).astype(jnp.int32)
    out = _build(B, L, V, D, NC, NS)(xf, token_table, pos_table)
    return out.reshape(B, L, D)


# speedup vs baseline: 1.3838x; 1.3133x over previous
import functools

import jax
import jax.numpy as jnp
from jax import lax
from jax.experimental import pallas as pl
from jax.experimental.pallas import tpu as pltpu
from jax.experimental.pallas import tpu_sc as plsc


def _build(B, L, V, D, NC, NS):
    NW = NC * NS
    ROWS = B * L
    rows_per_w = ROWS // NW
    C = 4 * L
    n_chunks = rows_per_w // C
    SUB = 80
    n_sub = C // SUB

    mesh = plsc.VectorSubcoreMesh(core_axis_name="c", subcore_axis_name="s")

    @functools.partial(
        pl.kernel,
        out_type=jax.ShapeDtypeStruct((ROWS, 2 * D), jnp.float32),
        mesh=mesh,
        compiler_params=pltpu.CompilerParams(use_tc_tiling_on_sc=False),
        scratch_types=[
            pltpu.VMEM((2, C), jnp.int32),
            pltpu.VMEM((2, C, D), jnp.float32),
            pltpu.VMEM_SHARED((C, D), jnp.float32),
            pltpu.SemaphoreType.DMA,
            pltpu.SemaphoreType.DMA,
            pltpu.SemaphoreType.DMA,
            pltpu.SemaphoreType.DMA,
        ],
    )
    def embed(idx_hbm, tok_hbm, pos_hbm, out_hbm,
              idx_v, buf_v, pat_sh, sem_g0, sem_g1, sem_w0, sem_w1):
        sem_g = (sem_g0, sem_g1)
        sem_w = (sem_w0, sem_w1)
        wid = lax.axis_index("s") * NC + lax.axis_index("c")
        wbase = wid * rows_per_w

        @pl.when(lax.axis_index("s") == 0)
        def _stage():
            pltpu.sync_copy(pos_hbm, buf_v.at[0, pl.ds(0, L)])
            for r in range(C // L):
                pltpu.sync_copy(buf_v.at[0, pl.ds(0, L)],
                                pat_sh.at[pl.ds(r * L, L)])

        plsc.subcore_barrier()

        def fire(i, slot):
            base = wbase + i * C
            pltpu.sync_copy(idx_hbm.at[pl.ds(base, C)], idx_v.at[slot])
            pltpu.sync_copy(pat_sh, buf_v.at[slot])
            for j in range(n_sub):
                pltpu.async_copy(
                    tok_hbm.at[idx_v.at[slot, pl.ds(j * SUB, SUB)]],
                    buf_v.at[slot, pl.ds(j * SUB, SUB)],
                    sem_g[slot], add=True)

        def wait_gathers(slot):
            for j in range(n_sub):
                pltpu.make_async_copy(
                    tok_hbm.at[idx_v.at[slot, pl.ds(j * SUB, SUB)]],
                    buf_v.at[slot, pl.ds(j * SUB, SUB)],
                    sem_g[slot]).wait()

        def wb_wait(i, slot):
            pltpu.make_async_copy(
                buf_v.at[slot],
                out_hbm.at[pl.ds(wbase + i * C, C), pl.ds(0, D)],
                sem_w[slot]).wait()

        fire(0, 0)

        @pl.loop(0, n_chunks // 2)
        def _groups(g):
            for b in (0, 1):
                i = 2 * g + b
                nb = (b + 1) % 2

                @pl.when(i + 1 < n_chunks)
                def _next():
                    @pl.when(i >= 1)
                    def _drain():
                        wb_wait(i - 1, nb)
                    fire(i + 1, nb)

                wait_gathers(b)
                pltpu.async_copy(
                    buf_v.at[b],
                    out_hbm.at[pl.ds(wbase + i * C, C), pl.ds(0, D)],
                    sem_w[b])

        for slot, i in ((0, n_chunks - 2), (1, n_chunks - 1)):
            wb_wait(i, slot)

    return embed


def kernel(x, token_table, pos_table):
    B, L = x.shape
    V, D = token_table.shape
    try:
        info = plsc.get_sparse_core_info()
        NC, NS = info.num_cores, info.num_subcores
    except Exception:
        NC, NS = 2, 16
    xf = x.reshape(-1).astype(jnp.int32)
    out = _build(B, L, V, D, NC, NS)(xf, token_table, pos_table)
    return out[:, :D].reshape(B, L, D)


# speedup vs baseline: 1.4714x; 1.0633x over previous
import functools

import jax
import jax.numpy as jnp
from jax import lax
from jax.experimental import pallas as pl
from jax.experimental.pallas import tpu as pltpu
from jax.experimental.pallas import tpu_sc as plsc


def _build(B, L, V, D, NC, NS):
    NW = NC * NS
    ROWS = B * L
    rows_per_w = ROWS // NW
    C = 2 * L
    n_chunks = rows_per_w // C
    SUB = 80
    n_sub = C // SUB
    NSLOT = 3

    mesh = plsc.VectorSubcoreMesh(core_axis_name="c", subcore_axis_name="s")

    @functools.partial(
        pl.kernel,
        out_type=jax.ShapeDtypeStruct((ROWS, 2 * D), jnp.float32),
        mesh=mesh,
        compiler_params=pltpu.CompilerParams(use_tc_tiling_on_sc=False),
        scratch_types=[
            pltpu.VMEM((NSLOT, C), jnp.int32),
            pltpu.VMEM((NSLOT, C, D), jnp.float32),
            pltpu.VMEM_SHARED((C, D), jnp.float32),
        ] + [pltpu.SemaphoreType.DMA] * (3 * NSLOT),
    )
    def embed(idx_hbm, tok_hbm, pos_hbm, out_hbm,
              idx_v, buf_v, pat_sh, *sems):
        sem_g = sems[0:NSLOT]
        sem_w = sems[NSLOT:2 * NSLOT]
        sem_p = sems[2 * NSLOT:3 * NSLOT]
        wid = lax.axis_index("s") * NC + lax.axis_index("c")
        wbase = wid * rows_per_w

        @pl.when(lax.axis_index("s") == 0)
        def _stage():
            pltpu.sync_copy(pos_hbm, buf_v.at[0, pl.ds(0, L)])
            for r in range(C // L):
                pltpu.sync_copy(buf_v.at[0, pl.ds(0, L)],
                                pat_sh.at[pl.ds(r * L, L)])

        plsc.subcore_barrier()

        def prefill_start(slot):
            pltpu.async_copy(pat_sh, buf_v.at[slot], sem_p[slot])

        def prefill_wait(slot):
            pltpu.make_async_copy(pat_sh, buf_v.at[slot], sem_p[slot]).wait()

        def idx_load(i, slot):
            pltpu.sync_copy(idx_hbm.at[pl.ds(wbase + i * C, C)],
                            idx_v.at[slot])

        def gathers_start(slot):
            for j in range(n_sub):
                pltpu.async_copy(
                    tok_hbm.at[idx_v.at[slot, pl.ds(j * SUB, SUB)]],
                    buf_v.at[slot, pl.ds(j * SUB, SUB)],
                    sem_g[slot], add=True)

        def gathers_wait(slot):
            for j in range(n_sub):
                pltpu.make_async_copy(
                    tok_hbm.at[idx_v.at[slot, pl.ds(j * SUB, SUB)]],
                    buf_v.at[slot, pl.ds(j * SUB, SUB)],
                    sem_g[slot]).wait()

        def wb_start(i, slot):
            pltpu.async_copy(
                buf_v.at[slot],
                out_hbm.at[pl.ds(wbase + i * C, C), pl.ds(0, D)],
                sem_w[slot])

        def wb_wait(i, slot):
            pltpu.make_async_copy(
                buf_v.at[slot],
                out_hbm.at[pl.ds(wbase + i * C, C), pl.ds(0, D)],
                sem_w[slot]).wait()

        idx_load(0, 0)
        prefill_start(0)
        prefill_wait(0)
        gathers_start(0)
        idx_load(1, 1)
        prefill_start(1)

        def step(i, s0, s1, s2):
            @pl.when(i + 2 < n_chunks)
            def _prep():
                idx_load(i + 2, s2)

                @pl.when(i >= 1)
                def _drain():
                    wb_wait(i - 1, s2)

                prefill_start(s2)

            @pl.when(i + 1 < n_chunks)
            def _fire():
                prefill_wait(s1)
                gathers_start(s1)

            gathers_wait(s0)
            wb_start(i, s0)

        n_loop = (n_chunks // NSLOT) * NSLOT

        @pl.loop(0, n_loop // NSLOT)
        def _groups(g):
            for b in range(NSLOT):
                i = NSLOT * g + b
                step(i, b, (b + 1) % NSLOT, (b + 2) % NSLOT)

        for i in range(n_loop, n_chunks):
            s = i % NSLOT
            gathers_wait(s)
            wb_start(i, s)

        for i in (n_chunks - 3, n_chunks - 2, n_chunks - 1):
            wb_wait(i, i % NSLOT)

    return embed


def kernel(x, token_table, pos_table):
    B, L = x.shape
    V, D = token_table.shape
    try:
        info = plsc.get_sparse_core_info()
        NC, NS = info.num_cores, info.num_subcores
    except Exception:
        NC, NS = 2, 16
    xf = x.reshape(-1).astype(jnp.int32)
    out = _build(B, L, V, D, NC, NS)(xf, token_table, pos_table)
    return out[:, :D].reshape(B, L, D)


# speedup vs baseline: 1.4720x; 1.0004x over previous
import functools

import jax
import jax.numpy as jnp
from jax import lax
from jax.experimental import pallas as pl
from jax.experimental.pallas import tpu as pltpu
from jax.experimental.pallas import tpu_sc as plsc


def _build(B, L, V, D, NC, NS):
    NW = NC * NS
    ROWS = B * L
    rows_per_w = ROWS // NW
    C = 2 * L
    n_chunks = rows_per_w // C
    SUBS = ((0, 128), (128, 128), (256, 128), (384, 16))
    NSLOT = 4

    mesh = plsc.VectorSubcoreMesh(core_axis_name="c", subcore_axis_name="s")

    @functools.partial(
        pl.kernel,
        out_type=jax.ShapeDtypeStruct((ROWS, 2 * D), jnp.float32),
        mesh=mesh,
        compiler_params=pltpu.CompilerParams(use_tc_tiling_on_sc=False),
        scratch_types=[
            pltpu.VMEM((NSLOT, C), jnp.int32),
            pltpu.VMEM((NSLOT, C, D), jnp.float32),
            pltpu.VMEM_SHARED((C, D), jnp.float32),
        ] + [pltpu.SemaphoreType.DMA] * (4 * NSLOT),
    )
    def embed(idx_hbm, tok_hbm, pos_hbm, out_hbm,
              idx_v, buf_v, pat_sh, *sems):
        sem_g = sems[0:NSLOT]
        sem_w = sems[NSLOT:2 * NSLOT]
        sem_p = sems[2 * NSLOT:3 * NSLOT]
        sem_i = sems[3 * NSLOT:4 * NSLOT]
        wid = lax.axis_index("s") * NC + lax.axis_index("c")
        wbase = wid * rows_per_w

        @pl.when(lax.axis_index("s") == 0)
        def _stage():
            pltpu.sync_copy(pos_hbm, buf_v.at[0, pl.ds(0, L)])
            for r in range(C // L):
                pltpu.sync_copy(buf_v.at[0, pl.ds(0, L)],
                                pat_sh.at[pl.ds(r * L, L)])

        plsc.subcore_barrier()

        def prefill_start(slot):
            pltpu.async_copy(pat_sh, buf_v.at[slot], sem_p[slot])

        def prefill_wait(slot):
            pltpu.make_async_copy(pat_sh, buf_v.at[slot], sem_p[slot]).wait()

        def idx_start(i, slot):
            pltpu.async_copy(idx_hbm.at[pl.ds(wbase + i * C, C)],
                             idx_v.at[slot], sem_i[slot])

        def idx_wait(i, slot):
            pltpu.make_async_copy(idx_hbm.at[pl.ds(wbase + i * C, C)],
                                  idx_v.at[slot], sem_i[slot]).wait()

        def gathers_start(slot):
            for s0, sn in SUBS:
                pltpu.async_copy(
                    tok_hbm.at[idx_v.at[slot, pl.ds(s0, sn)]],
                    buf_v.at[slot, pl.ds(s0, sn)],
                    sem_g[slot], add=True)

        def gathers_wait(slot):
            for s0, sn in SUBS:
                pltpu.make_async_copy(
                    tok_hbm.at[idx_v.at[slot, pl.ds(s0, sn)]],
                    buf_v.at[slot, pl.ds(s0, sn)],
                    sem_g[slot]).wait()

        def wb_start(i, slot):
            pltpu.async_copy(
                buf_v.at[slot],
                out_hbm.at[pl.ds(wbase + i * C, C), pl.ds(0, D)],
                sem_w[slot])

        def wb_wait(i, slot):
            pltpu.make_async_copy(
                buf_v.at[slot],
                out_hbm.at[pl.ds(wbase + i * C, C), pl.ds(0, D)],
                sem_w[slot]).wait()

        idx_start(0, 0)
        prefill_start(0)
        idx_wait(0, 0)
        prefill_wait(0)
        gathers_start(0)
        idx_start(1, 1)
        prefill_start(1)
        idx_start(2, 2)

        def step(i, s0, s1, s2, s3):
            @pl.when(i + 3 < n_chunks)
            def _idx():
                idx_start(i + 3, s3)

            @pl.when(i + 2 < n_chunks)
            def _prep():
                @pl.when(i >= 2)
                def _drain():
                    wb_wait(i - 2, s2)

                prefill_start(s2)

            @pl.when(i + 1 < n_chunks)
            def _fire():
                idx_wait(i + 1, s1)
                prefill_wait(s1)
                gathers_start(s1)

            gathers_wait(s0)
            wb_start(i, s0)

        @pl.loop(0, n_chunks // NSLOT)
        def _groups(g):
            for b in range(NSLOT):
                i = NSLOT * g + b
                step(i, b, (b + 1) % NSLOT, (b + 2) % NSLOT,
                     (b + 3) % NSLOT)

        for i in range(n_chunks - 4, n_chunks):
            wb_wait(i, i % NSLOT)

    return embed


def kernel(x, token_table, pos_table):
    B, L = x.shape
    V, D = token_table.shape
    try:
        info = plsc.get_sparse_core_info()
        NC, NS = info.num_cores, info.num_subcores
    except Exception:
        NC, NS = 2, 16
    xf = x.reshape(-1).astype(jnp.int32)
    out = _build(B, L, V, D, NC, NS)(xf, token_table, pos_table)
    return out[:, :D].reshape(B, L, D)
